# Initial kernel scaffold; baseline (speedup 1.0000x reference)
import jax
import jax.numpy as jnp
from jax.experimental import pallas as pl


def kernel(batch_seq_cat, lanes_tab, maxspeed_tab, length_tab, lon_tab, lat_tab, W, b):
    raise NotImplementedError("write your pallas kernel here")


# speedup vs baseline: 1.8077x; 1.8077x over previous
import functools

import jax
import jax.numpy as jnp
from jax import lax
from jax.experimental import pallas as pl
from jax.experimental.pallas import tpu as pltpu
from jax.experimental.pallas import tpu_sc as plsc

B = 16384
EMB = 32
HID = 128
NT = 5

NC = 2
NS = 16
NW = NC * NS
RW = B // NW
CHUNK = 128
NCH = RW // CHUNK


def _sc_gather_body(idx_hbm, t0, t1, t2, t3, t4, out_hbm, idx_v, rows_v, sem):
    tabs = (t0, t1, t2, t3, t4)
    c = lax.axis_index("c")
    s = lax.axis_index("s")
    wid = s * NC + c
    pltpu.sync_copy(idx_hbm.at[wid], idx_v)
    handles = []
    for t in range(NT):
        for j in range(NCH):
            handles.append(
                pltpu.async_copy(
                    tabs[t].at[idx_v.at[t, j]],
                    rows_v.at[t, pl.ds(j * CHUNK, CHUNK)],
                    sem,
                )
            )
    for h in handles:
        h.wait()
    base = wid * RW
    for t in range(NT):
        pltpu.sync_copy(rows_v.at[t], out_hbm.at[t, pl.ds(base, RW)])


def _sc_gather(idx_w, tabs):
    mesh = plsc.VectorSubcoreMesh(core_axis_name="c", subcore_axis_name="s")
    kfn = functools.partial(
        pl.kernel,
        out_type=jax.ShapeDtypeStruct((NT, B, EMB), jnp.float32),
        mesh=mesh,
        scratch_types=[
            pltpu.VMEM((NT, NCH, CHUNK), jnp.int32),
            pltpu.VMEM((NT, RW, EMB), jnp.float32),
            pltpu.SemaphoreType.DMA,
        ],
        compiler_params=pltpu.CompilerParams(use_tc_tiling_on_sc=False),
    )(_sc_gather_body)
    return kfn(idx_w, *tabs)


BB = 2048


def _mm_body(g_ref, wt_ref, b_ref, o_ref):
    acc = None
    for t in range(NT):
        part = jax.lax.dot_general(
            g_ref[t],
            wt_ref[t * EMB:(t + 1) * EMB, :],
            (((1,), (0,)), ((), ())),
            preferred_element_type=jnp.float32,
        )
        acc = part if acc is None else acc + part
    o_ref[...] = acc + b_ref[...]


def _tc_project(gathered, Wt, b2):
    return pl.pallas_call(
        _mm_body,
        grid=(B // BB,),
        in_specs=[
            pl.BlockSpec((NT, BB, EMB), lambda i: (0, i, 0)),
            pl.BlockSpec((NT * EMB, HID), lambda i: (0, 0)),
            pl.BlockSpec((1, HID), lambda i: (0, 0)),
        ],
        out_specs=pl.BlockSpec((BB, HID), lambda i: (i, 0)),
        out_shape=jax.ShapeDtypeStruct((B, HID), jnp.float32),
    )(gathered, Wt, b2)


def kernel(batch_seq_cat, lanes_tab, maxspeed_tab, length_tab, lon_tab, lat_tab, W, b):
    idx5 = batch_seq_cat[:, 1:6].astype(jnp.int32)
    idx_w = idx5.reshape(NW, RW, NT).transpose(0, 2, 1).reshape(NW, NT, NCH, CHUNK)
    gathered = _sc_gather(idx_w, (lanes_tab, maxspeed_tab, length_tab, lon_tab, lat_tab))
    Wt = W.T
    b2 = b.reshape(1, HID)
    return _tc_project(gathered, Wt, b2)


# speedup vs baseline: 3.5136x; 1.9436x over previous
import functools

import jax
import jax.numpy as jnp
from jax import lax
from jax.experimental import pallas as pl
from jax.experimental.pallas import tpu as pltpu
from jax.experimental.pallas import tpu_sc as plsc

B = 16384
EMB = 32
HID = 128
V = 100000
NT = 5

NC = 2
NS = 16
NW = NC * NS
RW = B // NW
CHUNK = 128
NCH = RW // CHUNK

CB = 2048
NBLK = (V + CB - 1) // CB


def _project_body(t0, t1, t2, t3, t4, w_ref, b_ref, o_ref):
    tabs = (t0, t1, t2, t3, t4)
    for t in range(NT):
        p = lax.dot_general(
            tabs[t][...].astype(jnp.bfloat16),
            w_ref[t].astype(jnp.bfloat16),
            (((0,), (0,)), ((), ())),
            preferred_element_type=jnp.float32,
        )
        if t == 0:
            p = p + b_ref[...]
        o_ref[t] = p


def _tc_project(tabsT, Wr, b2):
    return pl.pallas_call(
        _project_body,
        grid=(NBLK,),
        in_specs=[pl.BlockSpec((EMB, CB), lambda i: (0, i)) for _ in range(NT)]
        + [
            pl.BlockSpec((NT, EMB, HID), lambda i: (0, 0, 0)),
            pl.BlockSpec((1, HID), lambda i: (0, 0)),
        ],
        out_specs=pl.BlockSpec((NT, CB, HID), lambda i: (0, i, 0)),
        out_shape=jax.ShapeDtypeStruct((NT, V, HID), jnp.float32),
    )(*tabsT, Wr, b2)


def _sc_gather_body(idx_hbm, p_hbm, out_hbm, idx_v, acc_v, sem):
    c = lax.axis_index("c")
    s = lax.axis_index("s")
    wid = s * NC + c

    zrow = jnp.zeros((16,), jnp.float32)

    def _zero(i, _):
        for cc in range(HID // 16):
            acc_v[i, pl.ds(cc * 16, 16)] = zrow
        return 0

    lax.fori_loop(0, RW, _zero, 0)

    pltpu.sync_copy(idx_hbm.at[pl.ds(wid * NT * NCH, NT * NCH)], idx_v)

    handles = []
    for t in range(NT):
        for j in range(NCH):
            handles.append(
                pltpu.async_copy(
                    p_hbm.at[idx_v.at[t * NCH + j]],
                    acc_v.at[pl.ds(j * CHUNK, CHUNK)],
                    sem,
                    add=True,
                )
            )
    for h in handles:
        h.wait()
    pltpu.sync_copy(acc_v, out_hbm.at[pl.ds(wid * RW, RW)])


def _sc_gather_add(idx2, P2):
    mesh = plsc.VectorSubcoreMesh(core_axis_name="c", subcore_axis_name="s")
    kfn = functools.partial(
        pl.kernel,
        out_type=jax.ShapeDtypeStruct((B, HID), jnp.float32),
        mesh=mesh,
        scratch_types=[
            pltpu.VMEM((NT * NCH, CHUNK), jnp.int32),
            pltpu.VMEM((RW, HID), jnp.float32),
            pltpu.SemaphoreType.DMA,
        ],
        compiler_params=pltpu.CompilerParams(use_tc_tiling_on_sc=False),
    )(_sc_gather_body)
    return kfn(idx2, P2)


def kernel(batch_seq_cat, lanes_tab, maxspeed_tab, length_tab, lon_tab, lat_tab, W, b):
    idx5 = batch_seq_cat[:, 1:6].astype(jnp.int32) + jnp.arange(NT, dtype=jnp.int32)[None, :] * V
    idx2 = (
        idx5.reshape(NW, RW, NT)
        .transpose(0, 2, 1)
        .reshape(NW * NT * NCH, CHUNK)
    )
    tabsT = [t.T for t in (lanes_tab, maxspeed_tab, length_tab, lon_tab, lat_tab)]
    Wr = W.reshape(HID, NT, EMB).transpose(1, 2, 0)
    P = _tc_project(tabsT, Wr, b.reshape(1, HID))
    P2 = P.reshape(NT * V, HID)
    return _sc_gather_add(idx2, P2)


# speedup vs baseline: 3.6448x; 1.0373x over previous
import functools

import jax
import jax.numpy as jnp
from jax import lax
from jax.experimental import pallas as pl
from jax.experimental.pallas import tpu as pltpu
from jax.experimental.pallas import tpu_sc as plsc

B = 16384
EMB = 32
HID = 128
V = 100000
NT = 5

NC = 2
NS = 16
NW = NC * NS
RW = B // NW
CHUNK = 128
NCH = RW // CHUNK

CB = 4096
NBLK = (V + CB - 1) // CB


def _project_body(t0, t1, t2, t3, t4, w_ref, b_ref, o_ref):
    tabs = (t0, t1, t2, t3, t4)
    for t in range(NT):
        p = lax.dot_general(
            tabs[t][...].astype(jnp.bfloat16),
            w_ref[t].astype(jnp.bfloat16),
            (((0,), (0,)), ((), ())),
            preferred_element_type=jnp.float32,
        )
        if t == 0:
            p = p + b_ref[...]
        o_ref[t] = p


def _tc_project(tabsT, Wr, b2):
    return pl.pallas_call(
        _project_body,
        grid=(NBLK,),
        in_specs=[pl.BlockSpec((EMB, CB), lambda i: (0, i)) for _ in range(NT)]
        + [
            pl.BlockSpec((NT, EMB, HID), lambda i: (0, 0, 0)),
            pl.BlockSpec((1, HID), lambda i: (0, 0)),
        ],
        out_specs=pl.BlockSpec((NT, CB, HID), lambda i: (0, i, 0)),
        out_shape=jax.ShapeDtypeStruct((NT, V, HID), jnp.float32),
    )(*tabsT, Wr, b2)


def _sc_gather_body(idx_hbm, p_hbm, out_hbm, idx_v, acc_v, sem):
    c = lax.axis_index("c")
    s = lax.axis_index("s")
    wid = s * NC + c

    zrow = jnp.zeros((16,), jnp.float32)

    def _zero(i, _):
        for cc in range(HID // 16):
            acc_v[i, pl.ds(cc * 16, 16)] = zrow
        return 0

    lax.fori_loop(0, RW, _zero, 0)

    pltpu.sync_copy(idx_hbm.at[pl.ds(wid * NT * NCH, NT * NCH)], idx_v)

    handles = []
    for t in range(NT):
        for j in range(NCH):
            handles.append(
                pltpu.async_copy(
                    p_hbm.at[idx_v.at[t * NCH + j]],
                    acc_v.at[pl.ds(j * CHUNK, CHUNK)],
                    sem,
                    add=True,
                )
            )
    for h in handles:
        h.wait()
    pltpu.sync_copy(acc_v, out_hbm.at[pl.ds(wid * RW, RW)])


def _sc_gather_add(idx2, P2):
    mesh = plsc.VectorSubcoreMesh(core_axis_name="c", subcore_axis_name="s")
    kfn = functools.partial(
        pl.kernel,
        out_type=jax.ShapeDtypeStruct((B, HID), jnp.float32),
        mesh=mesh,
        scratch_types=[
            pltpu.VMEM((NT * NCH, CHUNK), jnp.int32),
            pltpu.VMEM((RW, HID), jnp.float32),
            pltpu.SemaphoreType.DMA,
        ],
        compiler_params=pltpu.CompilerParams(use_tc_tiling_on_sc=False),
    )(_sc_gather_body)
    return kfn(idx2, P2)


def kernel(batch_seq_cat, lanes_tab, maxspeed_tab, length_tab, lon_tab, lat_tab, W, b):
    idx5 = batch_seq_cat[:, 1:6].astype(jnp.int32) + jnp.arange(NT, dtype=jnp.int32)[None, :] * V
    idx2 = (
        idx5.reshape(NW, RW, NT)
        .transpose(0, 2, 1)
        .reshape(NW * NT * NCH, CHUNK)
    )
    tabsT = [t.T for t in (lanes_tab, maxspeed_tab, length_tab, lon_tab, lat_tab)]
    Wr = W.reshape(HID, NT, EMB).transpose(1, 2, 0)
    P = _tc_project(tabsT, Wr, b.reshape(1, HID))
    P2 = P.reshape(NT * V, HID)
    return _sc_gather_add(idx2, P2)


# speedup vs baseline: 3.6802x; 1.0097x over previous
import functools

import jax
import jax.numpy as jnp
from jax import lax
from jax.experimental import pallas as pl
from jax.experimental.pallas import tpu as pltpu
from jax.experimental.pallas import tpu_sc as plsc

B = 16384
EMB = 32
HID = 128
V = 100000
NT = 5

NC = 2
NS = 16
NW = NC * NS
RW = B // NW
CHUNK = 128
NCH = RW // CHUNK

CB = 8192
NBLK = (V + CB - 1) // CB


def _project_body(t0, t1, t2, t3, t4, w_ref, b_ref, o_ref):
    tabs = (t0, t1, t2, t3, t4)
    for t in range(NT):
        p = lax.dot_general(
            tabs[t][...].astype(jnp.bfloat16),
            w_ref[t].astype(jnp.bfloat16),
            (((0,), (0,)), ((), ())),
            preferred_element_type=jnp.float32,
        )
        if t == 0:
            p = p + b_ref[...]
        o_ref[t] = p


def _tc_project(tabsT, Wr, b2):
    return pl.pallas_call(
        _project_body,
        grid=(NBLK,),
        in_specs=[pl.BlockSpec((EMB, CB), lambda i: (0, i)) for _ in range(NT)]
        + [
            pl.BlockSpec((NT, EMB, HID), lambda i: (0, 0, 0)),
            pl.BlockSpec((1, HID), lambda i: (0, 0)),
        ],
        out_specs=pl.BlockSpec((NT, CB, HID), lambda i: (0, i, 0)),
        out_shape=jax.ShapeDtypeStruct((NT, V, HID), jnp.float32),
        compiler_params=pltpu.CompilerParams(vmem_limit_bytes=100 * 1024 * 1024),
    )(*tabsT, Wr, b2)


def _sc_gather_body(idx_hbm, p_hbm, out_hbm, idx_v, acc_v, sem):
    c = lax.axis_index("c")
    s = lax.axis_index("s")
    wid = s * NC + c

    zrow = jnp.zeros((16,), jnp.float32)

    def _zero(i, _):
        for cc in range(HID // 16):
            acc_v[i, pl.ds(cc * 16, 16)] = zrow
        return 0

    lax.fori_loop(0, RW, _zero, 0)

    pltpu.sync_copy(idx_hbm.at[pl.ds(wid * NT * NCH, NT * NCH)], idx_v)

    handles = []
    for t in range(NT):
        for j in range(NCH):
            handles.append(
                pltpu.async_copy(
                    p_hbm.at[idx_v.at[t * NCH + j]],
                    acc_v.at[pl.ds(j * CHUNK, CHUNK)],
                    sem,
                    add=True,
                )
            )
    for h in handles:
        h.wait()
    pltpu.sync_copy(acc_v, out_hbm.at[pl.ds(wid * RW, RW)])


def _sc_gather_add(idx2, P2):
    mesh = plsc.VectorSubcoreMesh(core_axis_name="c", subcore_axis_name="s")
    kfn = functools.partial(
        pl.kernel,
        out_type=jax.ShapeDtypeStruct((B, HID), jnp.float32),
        mesh=mesh,
        scratch_types=[
            pltpu.VMEM((NT * NCH, CHUNK), jnp.int32),
            pltpu.VMEM((RW, HID), jnp.float32),
            pltpu.SemaphoreType.DMA,
        ],
        compiler_params=pltpu.CompilerParams(use_tc_tiling_on_sc=False),
    )(_sc_gather_body)
    return kfn(idx2, P2)


def kernel(batch_seq_cat, lanes_tab, maxspeed_tab, length_tab, lon_tab, lat_tab, W, b):
    idx5 = batch_seq_cat[:, 1:6].astype(jnp.int32) + jnp.arange(NT, dtype=jnp.int32)[None, :] * V
    idx2 = (
        idx5.reshape(NW, RW, NT)
        .transpose(0, 2, 1)
        .reshape(NW * NT * NCH, CHUNK)
    )
    tabsT = [t.T for t in (lanes_tab, maxspeed_tab, length_tab, lon_tab, lat_tab)]
    Wr = W.reshape(HID, NT, EMB).transpose(1, 2, 0)
    P = _tc_project(tabsT, Wr, b.reshape(1, HID))
    P2 = P.reshape(NT * V, HID)
    return _sc_gather_add(idx2, P2)


# speedup vs baseline: 3.7199x; 1.0108x over previous
import functools

import jax
import jax.numpy as jnp
from jax import lax
from jax.experimental import pallas as pl
from jax.experimental.pallas import tpu as pltpu
from jax.experimental.pallas import tpu_sc as plsc

B = 16384
EMB = 32
HID = 128
V = 100000
NT = 5

NC = 2
NS = 16
NW = NC * NS
RW = B // NW
CHUNK = 128
NCH = RW // CHUNK

CB = 8192
NBLK = (V + CB - 1) // CB


def _project_body(t0, t1, t2, t3, t4, w_ref, b_ref, o_ref):
    tabs = (t0, t1, t2, t3, t4)
    for t in range(NT):
        p = lax.dot_general(
            tabs[t][...].astype(jnp.bfloat16),
            w_ref[t].astype(jnp.bfloat16),
            (((0,), (0,)), ((), ())),
            preferred_element_type=jnp.float32,
        )
        if t == 0:
            p = p + b_ref[...]
        o_ref[t] = p


def _tc_project(tabsT, Wr, b2):
    return pl.pallas_call(
        _project_body,
        grid=(NBLK,),
        in_specs=[pl.BlockSpec((EMB, CB), lambda i: (0, i)) for _ in range(NT)]
        + [
            pl.BlockSpec((NT, EMB, HID), lambda i: (0, 0, 0)),
            pl.BlockSpec((1, HID), lambda i: (0, 0)),
        ],
        out_specs=pl.BlockSpec((NT, CB, HID), lambda i: (0, i, 0)),
        out_shape=jax.ShapeDtypeStruct((NT, V, HID), jnp.float32),
        compiler_params=pltpu.CompilerParams(vmem_limit_bytes=100 * 1024 * 1024),
    )(*tabsT, Wr, b2)


def _sc_gather_body(idx_hbm, p_hbm, out_hbm, idx_v, acc_v, s0, s1, s2, s3, sw):
    c = lax.axis_index("c")
    s = lax.axis_index("s")
    wid = s * NC + c
    sems = (s0, s1, s2, s3)

    pltpu.sync_copy(idx_hbm.at[pl.ds(wid * NT * NCH, NT * NCH)], idx_v)

    def chunk_dst(j):
        return acc_v.at[pl.ds(j * CHUNK, CHUNK)]

    base_h = [
        pltpu.async_copy(p_hbm.at[idx_v.at[j]], chunk_dst(j), sems[j])
        for j in range(NCH)
    ]
    add_h = []
    for j in range(NCH):
        base_h[j].wait()
        add_h.append([
            pltpu.async_copy(
                p_hbm.at[idx_v.at[t * NCH + j]], chunk_dst(j), sems[j], add=True
            )
            for t in range(1, NT)
        ])
    out_h = []
    for j in range(NCH):
        for h in add_h[j]:
            h.wait()
        out_h.append(
            pltpu.async_copy(
                acc_v.at[pl.ds(j * CHUNK, CHUNK)],
                out_hbm.at[pl.ds(wid * RW + j * CHUNK, CHUNK)],
                sw,
            )
        )
    for h in out_h:
        h.wait()


def _sc_gather_add(idx2, P2):
    mesh = plsc.VectorSubcoreMesh(core_axis_name="c", subcore_axis_name="s")
    kfn = functools.partial(
        pl.kernel,
        out_type=jax.ShapeDtypeStruct((B, HID), jnp.float32),
        mesh=mesh,
        scratch_types=[
            pltpu.VMEM((NT * NCH, CHUNK), jnp.int32),
            pltpu.VMEM((RW, HID), jnp.float32),
            pltpu.SemaphoreType.DMA,
            pltpu.SemaphoreType.DMA,
            pltpu.SemaphoreType.DMA,
            pltpu.SemaphoreType.DMA,
            pltpu.SemaphoreType.DMA,
        ],
        compiler_params=pltpu.CompilerParams(use_tc_tiling_on_sc=False),
    )(_sc_gather_body)
    return kfn(idx2, P2)


def kernel(batch_seq_cat, lanes_tab, maxspeed_tab, length_tab, lon_tab, lat_tab, W, b):
    idx5 = batch_seq_cat[:, 1:6].astype(jnp.int32) + jnp.arange(NT, dtype=jnp.int32)[None, :] * V
    idx2 = (
        idx5.reshape(NW, RW, NT)
        .transpose(0, 2, 1)
        .reshape(NW * NT * NCH, CHUNK)
    )
    tabsT = [t.T for t in (lanes_tab, maxspeed_tab, length_tab, lon_tab, lat_tab)]
    Wr = W.reshape(HID, NT, EMB).transpose(1, 2, 0)
    P = _tc_project(tabsT, Wr, b.reshape(1, HID))
    P2 = P.reshape(NT * V, HID)
    return _sc_gather_add(idx2, P2)
